# baseline (speedup 1.0000x reference)
import functools

import jax
import jax.numpy as jnp
from jax import lax
from jax.experimental import pallas as pl
from jax.experimental.pallas import tpu as pltpu
from jax.experimental.pallas import tpu_sc as plsc

_N, _E, _D = 10000, 320000, 128
_NC, _NS = 2, 16
_NW = _NC * _NS
_C = 64
_MCH = 156
_TL = 16
_EM = _NW * _MCH * _C
_PASSES = (48, 48, 48, 12)
_RPT = 640
_NPAD = 10240
_DPT = _NPAD // _NS

_mesh = plsc.VectorSubcoreMesh(core_axis_name="c", subcore_axis_name="s")


@functools.partial(
    pl.kernel,
    mesh=_mesh,
    out_type=jax.ShapeDtypeStruct((2, _NPAD), jnp.float32),
    scratch_types=[
        pltpu.VMEM((_MCH, _C), jnp.int32),
        pltpu.VMEM((1, _TL), jnp.int32),
        pltpu.VMEM((_C,), jnp.float32),
        pltpu.VMEM((_DPT,), jnp.float32),
        pltpu.VMEM_SHARED((_NPAD,), jnp.float32),
        pltpu.SemaphoreType.DMA,
        pltpu.SemaphoreType.DMA,
    ],
)
def _sc_degree(dst_hbm, tdst_hbm, out_hbm, dsts_v, tdst_v, ones_v, zeros_v,
               acc_sh, sem_a, sem_b):
    c = lax.axis_index("c")
    s = lax.axis_index("s")
    wid = s * _NC + c
    for i in range(_C // 16):
        ones_v[pl.ds(i * 16, 16)] = jnp.ones((16,), jnp.float32)
    for i in range(_DPT // 16):
        zeros_v[pl.ds(i * 16, 16)] = jnp.zeros((16,), jnp.float32)
    pltpu.sync_copy(dst_hbm.at[wid], dsts_v)
    pltpu.sync_copy(tdst_hbm.at[wid], tdst_v)
    pltpu.sync_copy(zeros_v, acc_sh.at[pl.ds(s * _DPT, _DPT)])
    plsc.subcore_barrier()

    def _scat(j, sem):
        pltpu.async_copy(ones_v, acc_sh.at[dsts_v.at[j]], sem, add=True)

    def _wait(j, sem):
        pltpu.make_async_copy(ones_v, acc_sh.at[dsts_v.at[j]], sem).wait()

    _scat(0, sem_a)
    _scat(1, sem_b)

    def pair(jj, carry):
        j0 = 2 * jj
        _wait(j0 - 2, sem_a)
        _scat(j0, sem_a)
        _wait(j0 - 1, sem_b)
        _scat(j0 + 1, sem_b)
        return carry

    lax.fori_loop(1, _MCH // 2, pair, 0)
    _wait(_MCH - 2, sem_a)
    _wait(_MCH - 1, sem_b)
    pltpu.sync_copy(ones_v.at[pl.ds(0, _TL)], acc_sh.at[tdst_v.at[0]], add=True)
    plsc.subcore_barrier()
    pltpu.sync_copy(acc_sh.at[pl.ds(s * _DPT, _DPT)],
                    out_hbm.at[c, pl.ds(s * _DPT, _DPT)])


@functools.partial(
    pl.kernel,
    mesh=_mesh,
    out_type=jax.ShapeDtypeStruct((2, _N, _D), jnp.float32),
    scratch_types=[
        pltpu.VMEM((48, _C), jnp.int32),
        pltpu.VMEM((48, _C), jnp.int32),
        pltpu.VMEM((1, _TL), jnp.int32),
        pltpu.VMEM((1, _TL), jnp.int32),
        pltpu.VMEM((_C, _D), jnp.float32),
        pltpu.VMEM((_C, _D), jnp.float32),
        pltpu.VMEM((_C, _D), jnp.float32),
        pltpu.VMEM_SHARED((_N, _D), jnp.float32),
        pltpu.SemaphoreType.DMA,
        pltpu.SemaphoreType.DMA,
        pltpu.SemaphoreType.DMA,
        pltpu.SemaphoreType.DMA,
        pltpu.SemaphoreType.DMA,
        pltpu.SemaphoreType.DMA,
    ],
)
def _sc_agg(y_hbm, src_hbm, dst_hbm, tsrc_hbm, tdst_hbm, out_hbm,
            srcs_v, dsts_v, tsrc_v, tdst_v, rows_a, rows_b, rows_c, acc_sh,
            gsem_a, gsem_b, gsem_c, ssem_a, ssem_b, ssem_c):
    c = lax.axis_index("c")
    s = lax.axis_index("s")
    wid = s * _NC + c
    row0 = pl.multiple_of(jnp.minimum(s * _RPT, _N - _RPT), 8)
    pltpu.sync_copy(y_hbm.at[pl.ds(row0, _RPT)],
                    acc_sh.at[pl.ds(row0, _RPT)])
    pltpu.sync_copy(tsrc_hbm.at[wid], tsrc_v)
    pltpu.sync_copy(tdst_hbm.at[wid], tdst_v)
    plsc.subcore_barrier()

    bufs = (rows_a, rows_b, rows_c)
    gsems = (gsem_a, gsem_b, gsem_c)
    ssems = (ssem_a, ssem_b, ssem_c)

    def _gather(j, buf, sem):
        pltpu.async_copy(y_hbm.at[srcs_v.at[j]], buf, sem)

    def _wait_g(j, buf, sem):
        pltpu.make_async_copy(y_hbm.at[srcs_v.at[j]], buf, sem).wait()

    def _ascat(j, buf, sem):
        pltpu.async_copy(buf, acc_sh.at[dsts_v.at[j]], sem, add=True)

    def _wait_s(j, buf, sem):
        pltpu.make_async_copy(buf, acc_sh.at[dsts_v.at[j]], sem).wait()

    def _run_pass(k):
        for t in range(3):
            _gather(t, bufs[t], gsems[t])

        def triplet(tt, carry):
            j = 3 * tt
            for t in range(3):
                _wait_g(j + t, bufs[t], gsems[t])
                _ascat(j + t, bufs[t], ssems[t])
            for t in range(3):
                _wait_s(j + t, bufs[t], ssems[t])
                _gather(j + t + 3, bufs[t], gsems[t])
            return carry

        lax.fori_loop(0, k // 3 - 1, triplet, 0)
        j = k - 3
        for t in range(3):
            _wait_g(j + t, bufs[t], gsems[t])
            _ascat(j + t, bufs[t], ssems[t])
        for t in range(3):
            _wait_s(j + t, bufs[t], ssems[t])

    start = 0
    for k in _PASSES:
        pltpu.sync_copy(src_hbm.at[wid, pl.ds(start, k)],
                        srcs_v.at[pl.ds(0, k)])
        pltpu.sync_copy(dst_hbm.at[wid, pl.ds(start, k)],
                        dsts_v.at[pl.ds(0, k)])
        _run_pass(k)
        start += k

    tbuf = rows_a.at[pl.ds(0, _TL)]
    pltpu.async_copy(y_hbm.at[tsrc_v.at[0]], tbuf, gsem_a).wait()
    pltpu.sync_copy(tbuf, acc_sh.at[tdst_v.at[0]], add=True)

    plsc.subcore_barrier()
    pltpu.sync_copy(acc_sh.at[pl.ds(row0, _RPT)],
                    out_hbm.at[c, pl.ds(row0, _RPT)])


_R = 400
_G = _N // _R


def _d1_body(x_ref, w_ref, dinv_ref, y_ref):
    y_ref[...] = jnp.dot(x_ref[...], w_ref[...],
                         preferred_element_type=jnp.float32) * dinv_ref[...]


_dense1 = pl.pallas_call(
    _d1_body,
    grid=(_G,),
    in_specs=[
        pl.BlockSpec((_R, _D), lambda i: (i, 0)),
        pl.BlockSpec((_D, _D), lambda i: (0, 0)),
        pl.BlockSpec((_R, 1), lambda i: (i, 0)),
    ],
    out_specs=pl.BlockSpec((_R, _D), lambda i: (i, 0)),
    out_shape=jax.ShapeDtypeStruct((_N, _D), jnp.float32),
)


def _d2_body(p0_ref, p1_ref, y1_ref, dinv_ref, b1_ref, w2_ref, y2_ref):
    agg = p0_ref[0] + p1_ref[0] - y1_ref[...]
    h1 = jnp.maximum(agg * dinv_ref[...] + b1_ref[...], 0.0)
    y2_ref[...] = jnp.dot(h1, w2_ref[...],
                          preferred_element_type=jnp.float32) * dinv_ref[...]


_dense2 = pl.pallas_call(
    _d2_body,
    grid=(_G,),
    in_specs=[
        pl.BlockSpec((1, _R, _D), lambda i: (0, i, 0)),
        pl.BlockSpec((1, _R, _D), lambda i: (1, i, 0)),
        pl.BlockSpec((_R, _D), lambda i: (i, 0)),
        pl.BlockSpec((_R, 1), lambda i: (i, 0)),
        pl.BlockSpec((1, _D), lambda i: (0, 0)),
        pl.BlockSpec((_D, _D), lambda i: (0, 0)),
    ],
    out_specs=pl.BlockSpec((_R, _D), lambda i: (i, 0)),
    out_shape=jax.ShapeDtypeStruct((_N, _D), jnp.float32),
)


def _d3_body(q0_ref, q1_ref, y2_ref, dinv_ref, b2_ref, out_ref):
    h = (q0_ref[0] + q1_ref[0] - y2_ref[...]) * dinv_ref[...] + b2_ref[...]
    m = jnp.max(h, axis=1, keepdims=True)
    hm = h - m
    out_ref[...] = hm - jnp.log(jnp.sum(jnp.exp(hm), axis=1, keepdims=True))


_final = pl.pallas_call(
    _d3_body,
    grid=(_G,),
    in_specs=[
        pl.BlockSpec((1, _R, _D), lambda i: (0, i, 0)),
        pl.BlockSpec((1, _R, _D), lambda i: (1, i, 0)),
        pl.BlockSpec((_R, _D), lambda i: (i, 0)),
        pl.BlockSpec((_R, 1), lambda i: (i, 0)),
        pl.BlockSpec((1, _D), lambda i: (0, 0)),
    ],
    out_specs=pl.BlockSpec((_R, _D), lambda i: (i, 0)),
    out_shape=jax.ShapeDtypeStruct((_N, _D), jnp.float32),
)


def kernel(x, edge_index, W1, b1, W2, b2):
    e0 = edge_index[0]
    e1 = edge_index[1]
    src = e0[:_EM].reshape(_NW, _MCH, _C)
    dst = e1[:_EM].reshape(_NW, _MCH, _C)
    tsrc = e0[_EM:].reshape(_NW, 1, _TL)
    tdst = e1[_EM:].reshape(_NW, 1, _TL)
    degp = _sc_degree(dst, tdst)
    deg = degp[0, :_N] + degp[1, :_N] + 1.0
    dinv = lax.rsqrt(deg)[:, None]
    y1 = _dense1(x, W1, dinv)
    p = _sc_agg(y1, src, dst, tsrc, tdst)
    y2 = _dense2(p, p, y1, dinv, b1[None, :], W2)
    q = _sc_agg(y2, src, dst, tsrc, tdst)
    return _final(q, q, y2, dinv, b2[None, :])


# speedup vs baseline: 1.2072x; 1.2072x over previous
import functools

import jax
import jax.numpy as jnp
from jax import lax
from jax.experimental import pallas as pl
from jax.experimental.pallas import tpu as pltpu
from jax.experimental.pallas import tpu_sc as plsc

_N, _E, _D = 10000, 320000, 128
_NC, _NS = 2, 16
_NW = _NC * _NS
_C = 128
_MCH = 78
_TL = 16
_EM = _NW * _MCH * _C
_PASSES = (16, 16, 16, 16, 14)
_RPT = 640
_NPAD = 10240
_DPT = _NPAD // _NS

_mesh = plsc.VectorSubcoreMesh(core_axis_name="c", subcore_axis_name="s")


@functools.partial(
    pl.kernel,
    mesh=_mesh,
    out_type=jax.ShapeDtypeStruct((2, _NPAD), jnp.float32),
    scratch_types=[
        pltpu.VMEM((_MCH, _C), jnp.int32),
        pltpu.VMEM((1, _TL), jnp.int32),
        pltpu.VMEM((_C,), jnp.float32),
        pltpu.VMEM((_DPT,), jnp.float32),
        pltpu.VMEM_SHARED((_NPAD,), jnp.float32),
        pltpu.SemaphoreType.DMA,
        pltpu.SemaphoreType.DMA,
    ],
)
def _sc_degree(dst_hbm, tdst_hbm, out_hbm, dsts_v, tdst_v, ones_v, zeros_v,
               acc_sh, sem_a, sem_b):
    c = lax.axis_index("c")
    s = lax.axis_index("s")
    wid = s * _NC + c
    for i in range(_C // 16):
        ones_v[pl.ds(i * 16, 16)] = jnp.ones((16,), jnp.float32)
    for i in range(_DPT // 16):
        zeros_v[pl.ds(i * 16, 16)] = jnp.zeros((16,), jnp.float32)
    pltpu.sync_copy(dst_hbm.at[wid], dsts_v)
    pltpu.sync_copy(tdst_hbm.at[wid], tdst_v)
    pltpu.sync_copy(zeros_v, acc_sh.at[pl.ds(s * _DPT, _DPT)])
    plsc.subcore_barrier()

    def _scat(j, sem):
        pltpu.async_copy(ones_v, acc_sh.at[dsts_v.at[j]], sem, add=True)

    def _wait(j, sem):
        pltpu.make_async_copy(ones_v, acc_sh.at[dsts_v.at[j]], sem).wait()

    _scat(0, sem_a)
    _scat(1, sem_b)

    def pair(jj, carry):
        j0 = 2 * jj
        _wait(j0 - 2, sem_a)
        _scat(j0, sem_a)
        _wait(j0 - 1, sem_b)
        _scat(j0 + 1, sem_b)
        return carry

    lax.fori_loop(1, _MCH // 2, pair, 0)
    _wait(_MCH - 2, sem_a)
    _wait(_MCH - 1, sem_b)
    pltpu.sync_copy(ones_v.at[pl.ds(0, _TL)], acc_sh.at[tdst_v.at[0]], add=True)
    plsc.subcore_barrier()
    pltpu.sync_copy(acc_sh.at[pl.ds(s * _DPT, _DPT)],
                    out_hbm.at[c, pl.ds(s * _DPT, _DPT)])


@functools.partial(
    pl.kernel,
    mesh=_mesh,
    out_type=jax.ShapeDtypeStruct((2, _N, _D), jnp.float32),
    scratch_types=[
        pltpu.VMEM((16, _C), jnp.int32),
        pltpu.VMEM((16, _C), jnp.int32),
        pltpu.VMEM((1, _TL), jnp.int32),
        pltpu.VMEM((1, _TL), jnp.int32),
        pltpu.VMEM((_C, _D), jnp.float32),
        pltpu.VMEM((_C, _D), jnp.float32),
        pltpu.VMEM_SHARED((_N, _D), jnp.float32),
        pltpu.SemaphoreType.DMA,
        pltpu.SemaphoreType.DMA,
    ],
)
def _sc_agg(y_hbm, src_hbm, dst_hbm, tsrc_hbm, tdst_hbm, out_hbm,
            srcs_v, dsts_v, tsrc_v, tdst_v, rows_a, rows_b, acc_sh,
            sem_a, sem_b):
    c = lax.axis_index("c")
    s = lax.axis_index("s")
    wid = s * _NC + c
    row0 = pl.multiple_of(jnp.minimum(s * _RPT, _N - _RPT), 8)
    pltpu.sync_copy(y_hbm.at[pl.ds(row0, _RPT)],
                    acc_sh.at[pl.ds(row0, _RPT)])
    pltpu.sync_copy(tsrc_hbm.at[wid], tsrc_v)
    pltpu.sync_copy(tdst_hbm.at[wid], tdst_v)
    plsc.subcore_barrier()

    def _gather(j, buf, sem):
        pltpu.async_copy(y_hbm.at[srcs_v.at[j]], buf, sem)

    def _wait(j, buf, sem):
        pltpu.make_async_copy(y_hbm.at[srcs_v.at[j]], buf, sem).wait()

    def _scatter(j, buf):
        pltpu.sync_copy(buf, acc_sh.at[dsts_v.at[j]], add=True)

    def _run_pass(k):
        _gather(0, rows_a, sem_a)

        def pair(jj, carry):
            j0 = 2 * jj
            _gather(j0 + 1, rows_b, sem_b)
            _wait(j0, rows_a, sem_a)
            _scatter(j0, rows_a)
            _gather(j0 + 2, rows_a, sem_a)
            _wait(j0 + 1, rows_b, sem_b)
            _scatter(j0 + 1, rows_b)
            return carry

        lax.fori_loop(0, (k - 1) // 2, pair, 0)
        if k % 2 == 1:
            _wait(k - 1, rows_a, sem_a)
            _scatter(k - 1, rows_a)
        else:
            _gather(k - 1, rows_b, sem_b)
            _wait(k - 2, rows_a, sem_a)
            _scatter(k - 2, rows_a)
            _wait(k - 1, rows_b, sem_b)
            _scatter(k - 1, rows_b)

    start = 0
    for k in _PASSES:
        pltpu.sync_copy(src_hbm.at[wid, pl.ds(start, k)],
                        srcs_v.at[pl.ds(0, k)])
        pltpu.sync_copy(dst_hbm.at[wid, pl.ds(start, k)],
                        dsts_v.at[pl.ds(0, k)])
        _run_pass(k)
        start += k

    tbuf = rows_a.at[pl.ds(0, _TL)]
    pltpu.async_copy(y_hbm.at[tsrc_v.at[0]], tbuf, sem_a).wait()
    pltpu.sync_copy(tbuf, acc_sh.at[tdst_v.at[0]], add=True)

    plsc.subcore_barrier()
    pltpu.sync_copy(acc_sh.at[pl.ds(row0, _RPT)],
                    out_hbm.at[c, pl.ds(row0, _RPT)])


_R = 2000
_G = _N // _R


def _d1_body(x_ref, w_ref, deg_ref, y_ref):
    dinv = lax.rsqrt(deg_ref[0, 0])[:, None]
    y_ref[...] = jnp.dot(x_ref[...], w_ref[...],
                         preferred_element_type=jnp.float32) * dinv


_dense1 = pl.pallas_call(
    _d1_body,
    grid=(_G,),
    in_specs=[
        pl.BlockSpec((_R, _D), lambda i: (i, 0)),
        pl.BlockSpec((_D, _D), lambda i: (0, 0)),
        pl.BlockSpec((1, 1, _R), lambda i: (i, 0, 0)),
    ],
    out_specs=pl.BlockSpec((_R, _D), lambda i: (i, 0)),
    out_shape=jax.ShapeDtypeStruct((_N, _D), jnp.float32),
)


def _d2_body(p0_ref, p1_ref, y1_ref, deg_ref, b1_ref, w2_ref, y2_ref):
    dinv = lax.rsqrt(deg_ref[0, 0])[:, None]
    agg = p0_ref[0] + p1_ref[0] - y1_ref[...]
    h1 = jnp.maximum(agg * dinv + b1_ref[...], 0.0)
    y2_ref[...] = jnp.dot(h1, w2_ref[...],
                          preferred_element_type=jnp.float32) * dinv


_dense2 = pl.pallas_call(
    _d2_body,
    grid=(_G,),
    in_specs=[
        pl.BlockSpec((1, _R, _D), lambda i: (0, i, 0)),
        pl.BlockSpec((1, _R, _D), lambda i: (1, i, 0)),
        pl.BlockSpec((_R, _D), lambda i: (i, 0)),
        pl.BlockSpec((1, 1, _R), lambda i: (i, 0, 0)),
        pl.BlockSpec((1, _D), lambda i: (0, 0)),
        pl.BlockSpec((_D, _D), lambda i: (0, 0)),
    ],
    out_specs=pl.BlockSpec((_R, _D), lambda i: (i, 0)),
    out_shape=jax.ShapeDtypeStruct((_N, _D), jnp.float32),
)


def _d3_body(q0_ref, q1_ref, y2_ref, deg_ref, b2_ref, out_ref):
    dinv = lax.rsqrt(deg_ref[0, 0])[:, None]
    h = (q0_ref[0] + q1_ref[0] - y2_ref[...]) * dinv + b2_ref[...]
    m = jnp.max(h, axis=1, keepdims=True)
    hm = h - m
    out_ref[...] = hm - jnp.log(jnp.sum(jnp.exp(hm), axis=1, keepdims=True))


_final = pl.pallas_call(
    _d3_body,
    grid=(_G,),
    in_specs=[
        pl.BlockSpec((1, _R, _D), lambda i: (0, i, 0)),
        pl.BlockSpec((1, _R, _D), lambda i: (1, i, 0)),
        pl.BlockSpec((_R, _D), lambda i: (i, 0)),
        pl.BlockSpec((1, 1, _R), lambda i: (i, 0, 0)),
        pl.BlockSpec((1, _D), lambda i: (0, 0)),
    ],
    out_specs=pl.BlockSpec((_R, _D), lambda i: (i, 0)),
    out_shape=jax.ShapeDtypeStruct((_N, _D), jnp.float32),
)


def kernel(x, edge_index, W1, b1, W2, b2):
    e0 = edge_index[0]
    e1 = edge_index[1]
    src = e0[:_EM].reshape(_NW, _MCH, _C)
    dst = e1[:_EM].reshape(_NW, _MCH, _C)
    tsrc = e0[_EM:].reshape(_NW, 1, _TL)
    tdst = e1[_EM:].reshape(_NW, 1, _TL)
    degp = _sc_degree(dst, tdst)
    deg = (degp[0, :_N] + degp[1, :_N] + 1.0).reshape(_G, 1, _R)
    y1 = _dense1(x, W1, deg)
    p = _sc_agg(y1, src, dst, tsrc, tdst)
    y2 = _dense2(p, p, y1, deg, b1[None, :], W2)
    q = _sc_agg(y2, src, dst, tsrc, tdst)
    return _final(q, q, y2, deg, b2[None, :])


# speedup vs baseline: 1.3319x; 1.1033x over previous
import functools

import jax
import jax.numpy as jnp
from jax import lax
from jax.experimental import pallas as pl
from jax.experimental.pallas import tpu as pltpu
from jax.experimental.pallas import tpu_sc as plsc

_N, _E, _D = 10000, 320000, 128
_NC, _NS = 2, 16
_NW = _NC * _NS
_C = 128
_MCH = 78
_EM = _NW * _MCH * _C
_NX = (_E - _EM) // _C
_PASSES = (16, 16, 16, 16, 14)
_RPT = 640
_NPAD = 10240
_DPT = _NPAD // _NS

_mesh = plsc.VectorSubcoreMesh(core_axis_name="c", subcore_axis_name="s")


@functools.partial(
    pl.kernel,
    mesh=_mesh,
    out_type=jax.ShapeDtypeStruct((2, _NPAD), jnp.float32),
    scratch_types=[
        pltpu.VMEM((_MCH, _C), jnp.int32),
        pltpu.VMEM((1, _C), jnp.int32),
        pltpu.VMEM((_C,), jnp.float32),
        pltpu.VMEM((_DPT,), jnp.float32),
        pltpu.VMEM_SHARED((_NPAD,), jnp.float32),
        pltpu.SemaphoreType.DMA,
        pltpu.SemaphoreType.DMA,
    ],
)
def _sc_degree(dst_hbm, xdst_hbm, out_hbm, dsts_v, xdst_v, ones_v, zeros_v,
               acc_sh, sem_a, sem_b):
    c = lax.axis_index("c")
    s = lax.axis_index("s")
    wid = s * _NC + c
    for i in range(_C // 16):
        ones_v[pl.ds(i * 16, 16)] = jnp.ones((16,), jnp.float32)
    for i in range(_DPT // 16):
        zeros_v[pl.ds(i * 16, 16)] = jnp.zeros((16,), jnp.float32)
    pltpu.sync_copy(dst_hbm.at[wid], dsts_v)
    pltpu.sync_copy(zeros_v, acc_sh.at[pl.ds(s * _DPT, _DPT)])
    plsc.subcore_barrier()

    def _scat(j, sem):
        pltpu.async_copy(ones_v, acc_sh.at[dsts_v.at[j]], sem, add=True)

    def _wait(j, sem):
        pltpu.make_async_copy(ones_v, acc_sh.at[dsts_v.at[j]], sem).wait()

    _scat(0, sem_a)
    _scat(1, sem_b)

    def pair(jj, carry):
        j0 = 2 * jj
        _wait(j0 - 2, sem_a)
        _scat(j0, sem_a)
        _wait(j0 - 1, sem_b)
        _scat(j0 + 1, sem_b)
        return carry

    lax.fori_loop(1, _MCH // 2, pair, 0)
    _wait(_MCH - 2, sem_a)
    _wait(_MCH - 1, sem_b)

    @pl.when(wid < _NX)
    def _extra():
        pltpu.sync_copy(xdst_hbm.at[wid], xdst_v)
        pltpu.sync_copy(ones_v, acc_sh.at[xdst_v.at[0]], add=True)

    plsc.subcore_barrier()
    pltpu.sync_copy(acc_sh.at[pl.ds(s * _DPT, _DPT)],
                    out_hbm.at[c, pl.ds(s * _DPT, _DPT)])


@functools.partial(
    pl.kernel,
    mesh=_mesh,
    out_type=jax.ShapeDtypeStruct((2, _N, _D), jnp.float32),
    scratch_types=[
        pltpu.VMEM((16, _C), jnp.int32),
        pltpu.VMEM((16, _C), jnp.int32),
        pltpu.VMEM((1, _C), jnp.int32),
        pltpu.VMEM((1, _C), jnp.int32),
        pltpu.VMEM((_C, _D), jnp.float32),
        pltpu.VMEM((_C, _D), jnp.float32),
        pltpu.VMEM_SHARED((_N, _D), jnp.float32),
        pltpu.SemaphoreType.DMA,
        pltpu.SemaphoreType.DMA,
    ],
)
def _sc_agg(y_hbm, src_hbm, dst_hbm, xsrc_hbm, xdst_hbm, out_hbm,
            srcs_v, dsts_v, xsrc_v, xdst_v, rows_a, rows_b, acc_sh,
            sem_a, sem_b):
    c = lax.axis_index("c")
    s = lax.axis_index("s")
    wid = s * _NC + c
    row0 = pl.multiple_of(jnp.minimum(s * _RPT, _N - _RPT), 8)
    pltpu.sync_copy(y_hbm.at[pl.ds(row0, _RPT)],
                    acc_sh.at[pl.ds(row0, _RPT)])
    plsc.subcore_barrier()

    def _gather(j, buf, sem):
        pltpu.async_copy(y_hbm.at[srcs_v.at[j]], buf, sem)

    def _wait(j, buf, sem):
        pltpu.make_async_copy(y_hbm.at[srcs_v.at[j]], buf, sem).wait()

    def _scatter(j, buf):
        pltpu.sync_copy(buf, acc_sh.at[dsts_v.at[j]], add=True)

    def _run_pass(k):
        _gather(0, rows_a, sem_a)

        def pair(jj, carry):
            j0 = 2 * jj
            _gather(j0 + 1, rows_b, sem_b)
            _wait(j0, rows_a, sem_a)
            _scatter(j0, rows_a)
            _gather(j0 + 2, rows_a, sem_a)
            _wait(j0 + 1, rows_b, sem_b)
            _scatter(j0 + 1, rows_b)
            return carry

        lax.fori_loop(0, (k - 1) // 2, pair, 0)
        if k % 2 == 1:
            _wait(k - 1, rows_a, sem_a)
            _scatter(k - 1, rows_a)
        else:
            _gather(k - 1, rows_b, sem_b)
            _wait(k - 2, rows_a, sem_a)
            _scatter(k - 2, rows_a)
            _wait(k - 1, rows_b, sem_b)
            _scatter(k - 1, rows_b)

    start = 0
    for k in _PASSES:
        pltpu.sync_copy(src_hbm.at[wid, pl.ds(start, k)],
                        srcs_v.at[pl.ds(0, k)])
        pltpu.sync_copy(dst_hbm.at[wid, pl.ds(start, k)],
                        dsts_v.at[pl.ds(0, k)])
        _run_pass(k)
        start += k

    @pl.when(wid < _NX)
    def _extra():
        pltpu.sync_copy(xsrc_hbm.at[wid], xsrc_v)
        pltpu.sync_copy(xdst_hbm.at[wid], xdst_v)
        pltpu.async_copy(y_hbm.at[xsrc_v.at[0]], rows_a, sem_a).wait()
        pltpu.sync_copy(rows_a, acc_sh.at[xdst_v.at[0]], add=True)

    plsc.subcore_barrier()
    pltpu.sync_copy(acc_sh.at[pl.ds(row0, _RPT)],
                    out_hbm.at[c, pl.ds(row0, _RPT)])


def _ref_body(em_ref, ex_ref, src_ref, dst_ref, xsrc_ref, xdst_ref):
    src_ref[...] = em_ref[0].reshape(_NW, _MCH, _C)
    dst_ref[...] = em_ref[1].reshape(_NW, _MCH, _C)
    xsrc_ref[...] = ex_ref[0].reshape(_NX, 1, _C)
    xdst_ref[...] = ex_ref[1].reshape(_NX, 1, _C)


_reformat = pl.pallas_call(
    _ref_body,
    grid=(1,),
    in_specs=[
        pl.BlockSpec((2, _EM), lambda i: (0, 0)),
        pl.BlockSpec((2, _E - _EM), lambda i: (0, _EM // (_E - _EM))),
    ],
    out_specs=(
        pl.BlockSpec((_NW, _MCH, _C), lambda i: (0, 0, 0)),
        pl.BlockSpec((_NW, _MCH, _C), lambda i: (0, 0, 0)),
        pl.BlockSpec((_NX, 1, _C), lambda i: (0, 0, 0)),
        pl.BlockSpec((_NX, 1, _C), lambda i: (0, 0, 0)),
    ),
    out_shape=(
        jax.ShapeDtypeStruct((_NW, _MCH, _C), jnp.int32),
        jax.ShapeDtypeStruct((_NW, _MCH, _C), jnp.int32),
        jax.ShapeDtypeStruct((_NX, 1, _C), jnp.int32),
        jax.ShapeDtypeStruct((_NX, 1, _C), jnp.int32),
    ),
)


_R = 2000
_G = _N // _R


def _d1_body(x_ref, w_ref, deg_ref, y_ref):
    dinv = lax.rsqrt(deg_ref[0, 0])[:, None]
    y_ref[...] = jnp.dot(x_ref[...], w_ref[...],
                         preferred_element_type=jnp.float32) * dinv


_dense1 = pl.pallas_call(
    _d1_body,
    grid=(_G,),
    in_specs=[
        pl.BlockSpec((_R, _D), lambda i: (i, 0)),
        pl.BlockSpec((_D, _D), lambda i: (0, 0)),
        pl.BlockSpec((1, 1, _R), lambda i: (i, 0, 0)),
    ],
    out_specs=pl.BlockSpec((_R, _D), lambda i: (i, 0)),
    out_shape=jax.ShapeDtypeStruct((_N, _D), jnp.float32),
)


def _d2_body(p0_ref, p1_ref, y1_ref, deg_ref, b1_ref, w2_ref, y2_ref):
    dinv = lax.rsqrt(deg_ref[0, 0])[:, None]
    agg = p0_ref[0] + p1_ref[0] - y1_ref[...]
    h1 = jnp.maximum(agg * dinv + b1_ref[...], 0.0)
    y2_ref[...] = jnp.dot(h1, w2_ref[...],
                          preferred_element_type=jnp.float32) * dinv


_dense2 = pl.pallas_call(
    _d2_body,
    grid=(_G,),
    in_specs=[
        pl.BlockSpec((1, _R, _D), lambda i: (0, i, 0)),
        pl.BlockSpec((1, _R, _D), lambda i: (1, i, 0)),
        pl.BlockSpec((_R, _D), lambda i: (i, 0)),
        pl.BlockSpec((1, 1, _R), lambda i: (i, 0, 0)),
        pl.BlockSpec((1, _D), lambda i: (0, 0)),
        pl.BlockSpec((_D, _D), lambda i: (0, 0)),
    ],
    out_specs=pl.BlockSpec((_R, _D), lambda i: (i, 0)),
    out_shape=jax.ShapeDtypeStruct((_N, _D), jnp.float32),
)


def _d3_body(q0_ref, q1_ref, y2_ref, deg_ref, b2_ref, out_ref):
    dinv = lax.rsqrt(deg_ref[0, 0])[:, None]
    h = (q0_ref[0] + q1_ref[0] - y2_ref[...]) * dinv + b2_ref[...]
    m = jnp.max(h, axis=1, keepdims=True)
    hm = h - m
    out_ref[...] = hm - jnp.log(jnp.sum(jnp.exp(hm), axis=1, keepdims=True))


_final = pl.pallas_call(
    _d3_body,
    grid=(_G,),
    in_specs=[
        pl.BlockSpec((1, _R, _D), lambda i: (0, i, 0)),
        pl.BlockSpec((1, _R, _D), lambda i: (1, i, 0)),
        pl.BlockSpec((_R, _D), lambda i: (i, 0)),
        pl.BlockSpec((1, 1, _R), lambda i: (i, 0, 0)),
        pl.BlockSpec((1, _D), lambda i: (0, 0)),
    ],
    out_specs=pl.BlockSpec((_R, _D), lambda i: (i, 0)),
    out_shape=jax.ShapeDtypeStruct((_N, _D), jnp.float32),
)


def kernel(x, edge_index, W1, b1, W2, b2):
    src, dst, xsrc, xdst = _reformat(edge_index, edge_index)
    degp = _sc_degree(dst, xdst)
    deg = (degp[0, :_N] + degp[1, :_N] + 1.0).reshape(_G, 1, _R)
    y1 = _dense1(x, W1, deg)
    p = _sc_agg(y1, src, dst, xsrc, xdst)
    y2 = _dense2(p, p, y1, deg, b1[None, :], W2)
    q = _sc_agg(y2, src, dst, xsrc, xdst)
    return _final(q, q, y2, deg, b2[None, :])


# speedup vs baseline: 1.3731x; 1.0310x over previous
import functools

import jax
import jax.numpy as jnp
from jax import lax
from jax.experimental import pallas as pl
from jax.experimental.pallas import tpu as pltpu
from jax.experimental.pallas import tpu_sc as plsc

_N, _E, _D = 10000, 320000, 128
_NC, _NS = 2, 16
_NW = _NC * _NS
_C = 128
_MCH = 78
_MCHP = 80
_EM = _NW * _MCH * _C
_NX = (_E - _EM) // _C
_SLOT = 8
_RPT = 640
_NPAD = 10240
_DPT = _NPAD // _NS

_mesh = plsc.VectorSubcoreMesh(core_axis_name="c", subcore_axis_name="s")


@functools.partial(
    pl.kernel,
    mesh=_mesh,
    out_type=jax.ShapeDtypeStruct((2, _NPAD), jnp.float32),
    scratch_types=[
        pltpu.VMEM((_MCHP, _C), jnp.int32),
        pltpu.VMEM((1, _C), jnp.int32),
        pltpu.VMEM((_C,), jnp.float32),
        pltpu.VMEM((_DPT,), jnp.float32),
        pltpu.VMEM_SHARED((_NPAD,), jnp.float32),
        pltpu.SemaphoreType.DMA,
        pltpu.SemaphoreType.DMA,
    ],
)
def _sc_degree(dst_hbm, xdst_hbm, out_hbm, dsts_v, xdst_v, ones_v, zeros_v,
               acc_sh, sem_a, sem_b):
    c = lax.axis_index("c")
    s = lax.axis_index("s")
    wid = s * _NC + c
    for i in range(_C // 16):
        ones_v[pl.ds(i * 16, 16)] = jnp.ones((16,), jnp.float32)
    for i in range(_DPT // 16):
        zeros_v[pl.ds(i * 16, 16)] = jnp.zeros((16,), jnp.float32)
    pltpu.sync_copy(dst_hbm.at[wid], dsts_v)
    pltpu.sync_copy(zeros_v, acc_sh.at[pl.ds(s * _DPT, _DPT)])
    plsc.subcore_barrier()

    def _scat(j, sem):
        pltpu.async_copy(ones_v, acc_sh.at[dsts_v.at[j]], sem, add=True)

    def _wait(j, sem):
        pltpu.make_async_copy(ones_v, acc_sh.at[dsts_v.at[j]], sem).wait()

    _scat(0, sem_a)
    _scat(1, sem_b)

    def pair(jj, carry):
        j0 = 2 * jj
        _wait(j0 - 2, sem_a)
        _scat(j0, sem_a)
        _wait(j0 - 1, sem_b)
        _scat(j0 + 1, sem_b)
        return carry

    lax.fori_loop(1, _MCH // 2, pair, 0)
    _wait(_MCH - 2, sem_a)
    _wait(_MCH - 1, sem_b)

    @pl.when(wid < _NX)
    def _extra():
        pltpu.sync_copy(xdst_hbm.at[wid], xdst_v)
        pltpu.sync_copy(ones_v, acc_sh.at[xdst_v.at[0]], add=True)

    plsc.subcore_barrier()
    pltpu.sync_copy(acc_sh.at[pl.ds(s * _DPT, _DPT)],
                    out_hbm.at[c, pl.ds(s * _DPT, _DPT)])


@functools.partial(
    pl.kernel,
    mesh=_mesh,
    out_type=jax.ShapeDtypeStruct((2, _N, _D), jnp.float32),
    scratch_types=[
        pltpu.VMEM((2, _SLOT, _C), jnp.int32),
        pltpu.VMEM((2, _SLOT, _C), jnp.int32),
        pltpu.VMEM((1, _C), jnp.int32),
        pltpu.VMEM((1, _C), jnp.int32),
        pltpu.VMEM((_C, _D), jnp.float32),
        pltpu.VMEM((_C, _D), jnp.float32),
        pltpu.VMEM_SHARED((_N, _D), jnp.float32),
        pltpu.SemaphoreType.DMA,
        pltpu.SemaphoreType.DMA,
        pltpu.SemaphoreType.DMA,
    ],
)
def _sc_agg(y_hbm, src_hbm, dst_hbm, xsrc_hbm, xdst_hbm, out_hbm,
            srcs_v, dsts_v, xsrc_v, xdst_v, rows_a, rows_b, acc_sh,
            sem_a, sem_b, sem_i):
    c = lax.axis_index("c")
    s = lax.axis_index("s")
    wid = s * _NC + c
    row0 = pl.multiple_of(jnp.minimum(s * _RPT, _N - _RPT), 8)
    init = pltpu.async_copy(y_hbm.at[pl.ds(row0, _RPT)],
                            acc_sh.at[pl.ds(row0, _RPT)], sem_i)

    def _load_slot(slot, start):
        start = pl.multiple_of(start, 8)
        pltpu.sync_copy(src_hbm.at[wid, pl.ds(start, _SLOT)],
                        srcs_v.at[slot])
        pltpu.sync_copy(dst_hbm.at[wid, pl.ds(start, _SLOT)],
                        dsts_v.at[slot])

    def _src_row(j):
        return srcs_v.at[(j // _SLOT) % 2, lax.rem(j, _SLOT)]

    def _dst_row(j):
        return dsts_v.at[(j // _SLOT) % 2, lax.rem(j, _SLOT)]

    def _gather(j, buf, sem):
        pltpu.async_copy(y_hbm.at[_src_row(j)], buf, sem)

    def _wait(j, buf, sem):
        pltpu.make_async_copy(y_hbm.at[_src_row(j)], buf, sem).wait()

    def _scatter(j, buf):
        pltpu.sync_copy(buf, acc_sh.at[_dst_row(j)], add=True)

    _load_slot(0, 0)
    _gather(0, rows_a, sem_a)
    init.wait()
    plsc.subcore_barrier()

    def pair(jj, carry):
        j0 = 2 * jj

        @pl.when((lax.rem(j0, _SLOT) == 0) & (j0 + _SLOT < _MCHP))
        def _refill():
            _load_slot(((j0 // _SLOT) + 1) % 2, j0 + _SLOT)

        _gather(j0 + 1, rows_b, sem_b)
        _wait(j0, rows_a, sem_a)
        _scatter(j0, rows_a)
        _gather(j0 + 2, rows_a, sem_a)
        _wait(j0 + 1, rows_b, sem_b)
        _scatter(j0 + 1, rows_b)
        return carry

    lax.fori_loop(0, (_MCH - 2) // 2, pair, 0)
    _gather(_MCH - 1, rows_b, sem_b)
    _wait(_MCH - 2, rows_a, sem_a)
    _scatter(_MCH - 2, rows_a)
    _wait(_MCH - 1, rows_b, sem_b)
    _scatter(_MCH - 1, rows_b)

    @pl.when(wid < _NX)
    def _extra():
        pltpu.sync_copy(xsrc_hbm.at[wid], xsrc_v)
        pltpu.sync_copy(xdst_hbm.at[wid], xdst_v)
        pltpu.async_copy(y_hbm.at[xsrc_v.at[0]], rows_a, sem_a).wait()
        pltpu.sync_copy(rows_a, acc_sh.at[xdst_v.at[0]], add=True)

    plsc.subcore_barrier()
    pltpu.sync_copy(acc_sh.at[pl.ds(row0, _RPT)],
                    out_hbm.at[c, pl.ds(row0, _RPT)])


def _ref_body(em_ref, ex_ref, src_ref, dst_ref, xsrc_ref, xdst_ref):
    pad = jnp.zeros((_NW, _MCHP - _MCH, _C), jnp.int32)
    src_ref[...] = jnp.concatenate(
        [em_ref[0].reshape(_NW, _MCH, _C), pad], axis=1)
    dst_ref[...] = jnp.concatenate(
        [em_ref[1].reshape(_NW, _MCH, _C), pad], axis=1)
    xsrc_ref[...] = ex_ref[0].reshape(_NX, 1, _C)
    xdst_ref[...] = ex_ref[1].reshape(_NX, 1, _C)


_reformat = pl.pallas_call(
    _ref_body,
    grid=(1,),
    in_specs=[
        pl.BlockSpec((2, _EM), lambda i: (0, 0)),
        pl.BlockSpec((2, _E - _EM), lambda i: (0, _EM // (_E - _EM))),
    ],
    out_specs=(
        pl.BlockSpec((_NW, _MCHP, _C), lambda i: (0, 0, 0)),
        pl.BlockSpec((_NW, _MCHP, _C), lambda i: (0, 0, 0)),
        pl.BlockSpec((_NX, 1, _C), lambda i: (0, 0, 0)),
        pl.BlockSpec((_NX, 1, _C), lambda i: (0, 0, 0)),
    ),
    out_shape=(
        jax.ShapeDtypeStruct((_NW, _MCHP, _C), jnp.int32),
        jax.ShapeDtypeStruct((_NW, _MCHP, _C), jnp.int32),
        jax.ShapeDtypeStruct((_NX, 1, _C), jnp.int32),
        jax.ShapeDtypeStruct((_NX, 1, _C), jnp.int32),
    ),
)


_R = 2000
_G = _N // _R


def _d1_body(x_ref, w_ref, deg_ref, y_ref):
    dinv = lax.rsqrt(deg_ref[0, 0])[:, None]
    y_ref[...] = jnp.dot(x_ref[...], w_ref[...],
                         preferred_element_type=jnp.float32) * dinv


_dense1 = pl.pallas_call(
    _d1_body,
    grid=(_G,),
    in_specs=[
        pl.BlockSpec((_R, _D), lambda i: (i, 0)),
        pl.BlockSpec((_D, _D), lambda i: (0, 0)),
        pl.BlockSpec((1, 1, _R), lambda i: (i, 0, 0)),
    ],
    out_specs=pl.BlockSpec((_R, _D), lambda i: (i, 0)),
    out_shape=jax.ShapeDtypeStruct((_N, _D), jnp.float32),
)


def _d2_body(p0_ref, p1_ref, y1_ref, deg_ref, b1_ref, w2_ref, y2_ref):
    dinv = lax.rsqrt(deg_ref[0, 0])[:, None]
    agg = p0_ref[0] + p1_ref[0] - y1_ref[...]
    h1 = jnp.maximum(agg * dinv + b1_ref[...], 0.0)
    y2_ref[...] = jnp.dot(h1, w2_ref[...],
                          preferred_element_type=jnp.float32) * dinv


_dense2 = pl.pallas_call(
    _d2_body,
    grid=(_G,),
    in_specs=[
        pl.BlockSpec((1, _R, _D), lambda i: (0, i, 0)),
        pl.BlockSpec((1, _R, _D), lambda i: (1, i, 0)),
        pl.BlockSpec((_R, _D), lambda i: (i, 0)),
        pl.BlockSpec((1, 1, _R), lambda i: (i, 0, 0)),
        pl.BlockSpec((1, _D), lambda i: (0, 0)),
        pl.BlockSpec((_D, _D), lambda i: (0, 0)),
    ],
    out_specs=pl.BlockSpec((_R, _D), lambda i: (i, 0)),
    out_shape=jax.ShapeDtypeStruct((_N, _D), jnp.float32),
)


def _d3_body(q0_ref, q1_ref, y2_ref, deg_ref, b2_ref, out_ref):
    dinv = lax.rsqrt(deg_ref[0, 0])[:, None]
    h = (q0_ref[0] + q1_ref[0] - y2_ref[...]) * dinv + b2_ref[...]
    m = jnp.max(h, axis=1, keepdims=True)
    hm = h - m
    out_ref[...] = hm - jnp.log(jnp.sum(jnp.exp(hm), axis=1, keepdims=True))


_final = pl.pallas_call(
    _d3_body,
    grid=(_G,),
    in_specs=[
        pl.BlockSpec((1, _R, _D), lambda i: (0, i, 0)),
        pl.BlockSpec((1, _R, _D), lambda i: (1, i, 0)),
        pl.BlockSpec((_R, _D), lambda i: (i, 0)),
        pl.BlockSpec((1, 1, _R), lambda i: (i, 0, 0)),
        pl.BlockSpec((1, _D), lambda i: (0, 0)),
    ],
    out_specs=pl.BlockSpec((_R, _D), lambda i: (i, 0)),
    out_shape=jax.ShapeDtypeStruct((_N, _D), jnp.float32),
)


def kernel(x, edge_index, W1, b1, W2, b2):
    src, dst, xsrc, xdst = _reformat(edge_index, edge_index)
    degp = _sc_degree(dst, xdst)
    deg = (degp[0, :_N] + degp[1, :_N] + 1.0).reshape(_G, 1, _R)
    y1 = _dense1(x, W1, deg)
    p = _sc_agg(y1, src, dst, xsrc, xdst)
    y2 = _dense2(p, p, y1, deg, b1[None, :], W2)
    q = _sc_agg(y2, src, dst, xsrc, xdst)
    return _final(q, q, y2, deg, b2[None, :])
